# baseline (speedup 1.0000x reference)
import functools

import jax
import jax.numpy as jnp
from jax import lax
from jax.experimental import pallas as pl
from jax.experimental.pallas import tpu as pltpu
from jax.experimental.pallas import tpu_sc as plsc

_R2 = 0.2 * 0.2
_NS = 32
_MB = 128
_NC = 512
_DR = 8
_GB = 512


def _idx_body(qT_ref, xyzn_ref, dig_ref, out_ref):
    xn = xyzn_ref[0]
    qT = qT_ref[0]
    dig = dig_ref[0]
    n_total = xn.shape[0]

    ii = lax.broadcasted_iota(jnp.int32, (_NC, _NC), 0)
    jj = lax.broadcasted_iota(jnp.int32, (_NC, _NC), 1)
    ltri = (ii >= jj).astype(jnp.float8_e4m3fn)

    qx = qT[0:1, :]
    qy = qT[1:2, :]
    qz = qT[2:3, :]
    sqq = (qx * qx + qy * qy) + qz * qz
    qTb = qT.astype(jnp.bfloat16)
    xnb = xn.astype(jnp.bfloat16)

    acc = jnp.zeros((_DR, _NS * _MB), jnp.float32)
    k0 = jnp.zeros((1, _MB), jnp.float32)
    for ci in range(n_total // _NC):
        sl = slice(ci * _NC, (ci + 1) * _NC)
        px = xn[sl, 0:1]
        py = xn[sl, 1:2]
        pz = xn[sl, 2:3]
        sqx = (px * px + py * py) + pz * pz
        qp = jnp.dot(xnb[sl, :], qTb,
                     preferred_element_type=jnp.float32)
        dist = (sqq + sqx) - 2.0 * qp
        mk = dist < _R2
        mkb = jnp.where(mk, 1.0, 0.0).astype(jnp.float8_e4m3fn)
        c = jnp.dot(ltri, mkb, preferred_element_type=jnp.float32)
        s = jnp.where(mk, c + k0, 0.0)
        s_bf = jnp.minimum(s, 48.0).astype(jnp.bfloat16)
        one_b = jnp.bfloat16(1)
        zero_b = jnp.bfloat16(0)
        es = [jnp.where(s_bf == jnp.bfloat16(k), one_b,
                        zero_b).astype(jnp.float8_e4m3fn)
              for k in range(1, _NS + 1)]
        e_all = jnp.concatenate(es, axis=1)
        acc = acc + jnp.dot(dig[:, sl], e_all,
                            preferred_element_type=jnp.float32)
        k0 = k0 + c[_NC - 1:_NC, :]

    idxf = ((acc[0:1, :] * 4096.0 + acc[1:2, :] * 256.0)
            + (acc[2:3, :] * 16.0 + acc[3:4, :]))
    off = (pl.program_id(0) * n_total).astype(jnp.float32)
    idxf = idxf + off
    fillv = jnp.where(k0 > 0.5, idxf[:, 0:_MB], off)
    for k in range(1, _NS + 1):
        seg = idxf[:, (k - 1) * _MB:k * _MB]
        seg = jnp.where(k0 >= float(k), seg, fillv)
        out_ref[0, k - 1:k, :] = seg.astype(jnp.int32)


def _ball_query_idx(qT, xyz, dig):
    B, N, _ = xyz.shape
    M = qT.shape[2]
    return pl.pallas_call(
        _idx_body,
        grid=(B, M // _MB),
        in_specs=[
            pl.BlockSpec((1, 3, _MB), lambda b, mb: (b, 0, mb)),
            pl.BlockSpec((1, N, 3), lambda b, mb: (b, 0, 0)),
            pl.BlockSpec((1, _DR, N), lambda b, mb: (0, 0, 0)),
        ],
        out_specs=pl.BlockSpec((1, _NS, _MB), lambda b, mb: (b, 0, mb)),
        out_shape=jax.ShapeDtypeStruct((B, _NS, M), jnp.int32),
    )(qT, xyz, dig)


def _sc_gather(tbl, idxf):
    rows, d = idxf.shape[0], tbl.shape[1]
    info = plsc.get_sparse_core_info()
    nw = info.num_cores * info.num_subcores
    per_w = rows // nw
    nit = per_w // _GB
    mesh = plsc.VectorSubcoreMesh(core_axis_name="c", subcore_axis_name="s")

    @functools.partial(
        pl.kernel, mesh=mesh,
        out_type=jax.ShapeDtypeStruct((rows, d), jnp.float32),
        scratch_types=[
            pltpu.VMEM((_GB,), jnp.int32),
            pltpu.VMEM((_GB, d), jnp.float32),
            pltpu.SemaphoreType.DMA,
        ],
    )
    def gather_k(tbl_hbm, idx_hbm, out_hbm, idx_v, rows_v, sem):
        wid = lax.axis_index("s") * info.num_cores + lax.axis_index("c")
        base = wid * per_w

        def body(i, carry):
            off = base + i * _GB
            pltpu.sync_copy(idx_hbm.at[pl.ds(off, _GB)], idx_v)
            pltpu.async_copy(tbl_hbm.at[idx_v], rows_v, sem).wait()
            pltpu.sync_copy(rows_v, out_hbm.at[pl.ds(off, _GB)])
            return carry

        lax.fori_loop(0, nit, body, 0)

    return gather_k(tbl, idxf)


def kernel(xyz, new_xyz, features):
    B, N, _ = xyz.shape
    M = new_xyz.shape[1]
    C = features.shape[1]
    cp = 128
    tbl = jnp.concatenate(
        [xyz, jnp.transpose(features, (0, 2, 1)),
         jnp.zeros((B, N, cp - 3 - C), jnp.float32)], axis=2)
    tbl = tbl.reshape(B * N, cp)
    qT = jnp.transpose(new_xyz, (0, 2, 1))

    n_ar = jnp.arange(N, dtype=jnp.int32)
    dig = jnp.stack([(n_ar >> 12) & 15, (n_ar >> 8) & 15,
                     (n_ar >> 4) & 15, n_ar & 15], axis=0)
    dig = jnp.concatenate([dig, jnp.zeros((_DR - 4, N), jnp.int32)], axis=0)
    dig = dig.astype(jnp.float8_e4m3fn)[None]

    hb = B // 2
    tbl = tbl.reshape(2, hb * N, cp)
    outs = []
    for h in range(2):
        sl = slice(h * hb, (h + 1) * hb)
        idx = _ball_query_idx(qT[sl], xyz[sl], dig)
        g = _sc_gather(tbl[h], idx.reshape(hb * _NS * M))
        g = g.reshape(hb, _NS, M, cp)
        full = jnp.transpose(g, (0, 3, 2, 1))
        xyzp = full[:, 0:3] - qT[sl, :, :, None]
        outs.append(jnp.concatenate([xyzp, full[:, 3:3 + C]], axis=1))
    return jnp.concatenate(outs, axis=0)


# speedup vs baseline: 1.0479x; 1.0479x over previous
import functools

import jax
import jax.numpy as jnp
from jax import lax
from jax.experimental import pallas as pl
from jax.experimental.pallas import tpu as pltpu
from jax.experimental.pallas import tpu_sc as plsc

_R2 = 0.2 * 0.2
_NS = 32
_MB = 128
_NC = 512
_DR = 8
_GB = 512


def _idx_body(qT_ref, xyzn_ref, dig_ref, out_ref):
    xn = xyzn_ref[0]
    qT = qT_ref[0]
    dig = dig_ref[0]
    n_total = xn.shape[0]

    ii = lax.broadcasted_iota(jnp.int32, (_NC, _NC), 0)
    jj = lax.broadcasted_iota(jnp.int32, (_NC, _NC), 1)
    ltri = (ii >= jj).astype(jnp.float8_e4m3fn)

    qx = qT[0:1, :]
    qy = qT[1:2, :]
    qz = qT[2:3, :]
    sqq = (qx * qx + qy * qy) + qz * qz
    qxb = qx.astype(jnp.bfloat16).astype(jnp.float32)
    qyb = qy.astype(jnp.bfloat16).astype(jnp.float32)
    qzb = qz.astype(jnp.bfloat16).astype(jnp.float32)

    acc = jnp.zeros((_DR, _NS * _MB), jnp.float32)
    k0 = jnp.zeros((1, _MB), jnp.float32)
    for ci in range(n_total // _NC):
        sl = slice(ci * _NC, (ci + 1) * _NC)
        px = xn[sl, 0:1]
        py = xn[sl, 1:2]
        pz = xn[sl, 2:3]
        sqx = (px * px + py * py) + pz * pz
        pxb = px.astype(jnp.bfloat16).astype(jnp.float32)
        pyb = py.astype(jnp.bfloat16).astype(jnp.float32)
        pzb = pz.astype(jnp.bfloat16).astype(jnp.float32)
        qp = (pxb * qxb + pyb * qyb) + pzb * qzb
        dist = (sqq + sqx) - 2.0 * qp
        mk = dist < _R2
        mkb = jnp.where(mk, 1.0, 0.0).astype(jnp.float8_e4m3fn)
        c = jnp.dot(ltri, mkb, preferred_element_type=jnp.float32)
        s = jnp.where(mk, c + k0, 0.0)
        s_bf = jnp.minimum(s, 48.0).astype(jnp.bfloat16)
        one_b = jnp.bfloat16(1)
        zero_b = jnp.bfloat16(0)
        es = [jnp.where(s_bf == jnp.bfloat16(k), one_b,
                        zero_b).astype(jnp.float8_e4m3fn)
              for k in range(1, _NS + 1)]
        e_all = jnp.concatenate(es, axis=1)
        acc = acc + jnp.dot(dig[:, sl], e_all,
                            preferred_element_type=jnp.float32)
        k0 = k0 + c[_NC - 1:_NC, :]

    idxf = ((acc[0:1, :] * 4096.0 + acc[1:2, :] * 256.0)
            + (acc[2:3, :] * 16.0 + acc[3:4, :]))
    off = (pl.program_id(0) * n_total).astype(jnp.float32)
    idxf = idxf + off
    fillv = jnp.where(k0 > 0.5, idxf[:, 0:_MB], off)
    for k in range(1, _NS + 1):
        seg = idxf[:, (k - 1) * _MB:k * _MB]
        seg = jnp.where(k0 >= float(k), seg, fillv)
        out_ref[0, k - 1:k, :] = seg.astype(jnp.int32)


def _ball_query_idx(qT, xyz, dig):
    B, N, _ = xyz.shape
    M = qT.shape[2]
    return pl.pallas_call(
        _idx_body,
        grid=(B, M // _MB),
        in_specs=[
            pl.BlockSpec((1, 3, _MB), lambda b, mb: (b, 0, mb)),
            pl.BlockSpec((1, N, 3), lambda b, mb: (b, 0, 0)),
            pl.BlockSpec((1, _DR, N), lambda b, mb: (0, 0, 0)),
        ],
        out_specs=pl.BlockSpec((1, _NS, _MB), lambda b, mb: (b, 0, mb)),
        out_shape=jax.ShapeDtypeStruct((B, _NS, M), jnp.int32),
    )(qT, xyz, dig)


def _sc_gather(tbl, idxf):
    rows, d = idxf.shape[0], tbl.shape[1]
    info = plsc.get_sparse_core_info()
    nw = info.num_cores * info.num_subcores
    per_w = rows // nw
    nit = per_w // _GB
    mesh = plsc.VectorSubcoreMesh(core_axis_name="c", subcore_axis_name="s")

    @functools.partial(
        pl.kernel, mesh=mesh,
        out_type=jax.ShapeDtypeStruct((rows, d), jnp.float32),
        scratch_types=[
            pltpu.VMEM((_GB,), jnp.int32),
            pltpu.VMEM((_GB, d), jnp.float32),
            pltpu.SemaphoreType.DMA,
        ],
    )
    def gather_k(tbl_hbm, idx_hbm, out_hbm, idx_v, rows_v, sem):
        wid = lax.axis_index("s") * info.num_cores + lax.axis_index("c")
        base = wid * per_w

        def body(i, carry):
            off = base + i * _GB
            pltpu.sync_copy(idx_hbm.at[pl.ds(off, _GB)], idx_v)
            pltpu.async_copy(tbl_hbm.at[idx_v], rows_v, sem).wait()
            pltpu.sync_copy(rows_v, out_hbm.at[pl.ds(off, _GB)])
            return carry

        lax.fori_loop(0, nit, body, 0)

    return gather_k(tbl, idxf)


def kernel(xyz, new_xyz, features):
    B, N, _ = xyz.shape
    M = new_xyz.shape[1]
    C = features.shape[1]
    cp = 128
    tbl = jnp.concatenate(
        [xyz, jnp.transpose(features, (0, 2, 1)),
         jnp.zeros((B, N, cp - 3 - C), jnp.float32)], axis=2)
    tbl = tbl.reshape(B * N, cp)
    qT = jnp.transpose(new_xyz, (0, 2, 1))

    n_ar = jnp.arange(N, dtype=jnp.int32)
    dig = jnp.stack([(n_ar >> 12) & 15, (n_ar >> 8) & 15,
                     (n_ar >> 4) & 15, n_ar & 15], axis=0)
    dig = jnp.concatenate([dig, jnp.zeros((_DR - 4, N), jnp.int32)], axis=0)
    dig = dig.astype(jnp.float8_e4m3fn)[None]

    nsplit = 4
    hb = B // nsplit
    tbl = tbl.reshape(nsplit, hb * N, cp)
    outs = []
    for h in range(nsplit):
        sl = slice(h * hb, (h + 1) * hb)
        idx = _ball_query_idx(qT[sl], xyz[sl], dig)
        g = _sc_gather(tbl[h], idx.reshape(hb * _NS * M))
        g = g.reshape(hb, _NS, M, cp)
        full = jnp.transpose(g, (0, 3, 2, 1))
        xyzp = full[:, 0:3] - qT[sl, :, :, None]
        outs.append(jnp.concatenate([xyzp, full[:, 3:3 + C]], axis=1))
    return jnp.concatenate(outs, axis=0)


# speedup vs baseline: 1.0689x; 1.0200x over previous
import functools

import jax
import jax.numpy as jnp
from jax import lax
from jax.experimental import pallas as pl
from jax.experimental.pallas import tpu as pltpu
from jax.experimental.pallas import tpu_sc as plsc

_R2 = 0.2 * 0.2
_NS = 32
_MB = 256
_NC = 512
_DR = 8
_GB = 512


def _idx_body(qT_ref, xyzn_ref, dig_ref, out_ref):
    xn = xyzn_ref[0]
    qT = qT_ref[0]
    dig = dig_ref[0]
    n_total = xn.shape[0]

    ii = lax.broadcasted_iota(jnp.int32, (_NC, _NC), 0)
    jj = lax.broadcasted_iota(jnp.int32, (_NC, _NC), 1)
    ltri = (ii >= jj).astype(jnp.float8_e4m3fn)

    qx = qT[0:1, :]
    qy = qT[1:2, :]
    qz = qT[2:3, :]
    sqq = (qx * qx + qy * qy) + qz * qz
    qxb = qx.astype(jnp.bfloat16).astype(jnp.float32)
    qyb = qy.astype(jnp.bfloat16).astype(jnp.float32)
    qzb = qz.astype(jnp.bfloat16).astype(jnp.float32)

    acc = jnp.zeros((_DR, _NS * _MB), jnp.float32)
    k0 = jnp.zeros((1, _MB), jnp.float32)
    for ci in range(n_total // _NC):
        sl = slice(ci * _NC, (ci + 1) * _NC)
        px = xn[sl, 0:1]
        py = xn[sl, 1:2]
        pz = xn[sl, 2:3]
        sqx = (px * px + py * py) + pz * pz
        pxb = px.astype(jnp.bfloat16).astype(jnp.float32)
        pyb = py.astype(jnp.bfloat16).astype(jnp.float32)
        pzb = pz.astype(jnp.bfloat16).astype(jnp.float32)
        qp = (pxb * qxb + pyb * qyb) + pzb * qzb
        dist = (sqq + sqx) - 2.0 * qp
        mk = dist < _R2
        mkb = jnp.where(mk, 1.0, 0.0).astype(jnp.float8_e4m3fn)
        c = jnp.dot(ltri, mkb, preferred_element_type=jnp.float32)
        s = jnp.where(mk, c + k0, 0.0)
        s_bf = jnp.minimum(s, 48.0).astype(jnp.bfloat16)
        one_b = jnp.bfloat16(1)
        zero_b = jnp.bfloat16(0)
        es = [jnp.where(s_bf == jnp.bfloat16(k), one_b,
                        zero_b).astype(jnp.float8_e4m3fn)
              for k in range(1, _NS + 1)]
        e_all = jnp.concatenate(es, axis=1)
        acc = acc + jnp.dot(dig[:, sl], e_all,
                            preferred_element_type=jnp.float32)
        k0 = k0 + c[_NC - 1:_NC, :]

    idxf = ((acc[0:1, :] * 4096.0 + acc[1:2, :] * 256.0)
            + (acc[2:3, :] * 16.0 + acc[3:4, :]))
    off = (pl.program_id(0) * n_total).astype(jnp.float32)
    idxf = idxf + off
    fillv = jnp.where(k0 > 0.5, idxf[:, 0:_MB], off)
    for k in range(1, _NS + 1):
        seg = idxf[:, (k - 1) * _MB:k * _MB]
        seg = jnp.where(k0 >= float(k), seg, fillv)
        out_ref[0, k - 1:k, :] = seg.astype(jnp.int32)


def _ball_query_idx(qT, xyz, dig):
    B, N, _ = xyz.shape
    M = qT.shape[2]
    return pl.pallas_call(
        _idx_body,
        grid=(B, M // _MB),
        in_specs=[
            pl.BlockSpec((1, 3, _MB), lambda b, mb: (b, 0, mb)),
            pl.BlockSpec((1, N, 3), lambda b, mb: (b, 0, 0)),
            pl.BlockSpec((1, _DR, N), lambda b, mb: (0, 0, 0)),
        ],
        out_specs=pl.BlockSpec((1, _NS, _MB), lambda b, mb: (b, 0, mb)),
        out_shape=jax.ShapeDtypeStruct((B, _NS, M), jnp.int32),
    )(qT, xyz, dig)


def _sc_gather(tbl, idxf):
    rows, d = idxf.shape[0], tbl.shape[1]
    info = plsc.get_sparse_core_info()
    nw = info.num_cores * info.num_subcores
    per_w = rows // nw
    nit = per_w // _GB
    mesh = plsc.VectorSubcoreMesh(core_axis_name="c", subcore_axis_name="s")

    @functools.partial(
        pl.kernel, mesh=mesh,
        out_type=jax.ShapeDtypeStruct((rows, d), jnp.float32),
        scratch_types=[
            pltpu.VMEM((_GB,), jnp.int32),
            pltpu.VMEM((_GB, d), jnp.float32),
            pltpu.SemaphoreType.DMA,
        ],
    )
    def gather_k(tbl_hbm, idx_hbm, out_hbm, idx_v, rows_v, sem):
        wid = lax.axis_index("s") * info.num_cores + lax.axis_index("c")
        base = wid * per_w

        def body(i, carry):
            off = base + i * _GB
            pltpu.sync_copy(idx_hbm.at[pl.ds(off, _GB)], idx_v)
            pltpu.async_copy(tbl_hbm.at[idx_v], rows_v, sem).wait()
            pltpu.sync_copy(rows_v, out_hbm.at[pl.ds(off, _GB)])
            return carry

        lax.fori_loop(0, nit, body, 0)

    return gather_k(tbl, idxf)


def kernel(xyz, new_xyz, features):
    B, N, _ = xyz.shape
    M = new_xyz.shape[1]
    C = features.shape[1]
    cp = 128
    tbl = jnp.concatenate(
        [xyz, jnp.transpose(features, (0, 2, 1)),
         jnp.zeros((B, N, cp - 3 - C), jnp.float32)], axis=2)
    tbl = tbl.reshape(B * N, cp)
    qT = jnp.transpose(new_xyz, (0, 2, 1))

    n_ar = jnp.arange(N, dtype=jnp.int32)
    dig = jnp.stack([(n_ar >> 12) & 15, (n_ar >> 8) & 15,
                     (n_ar >> 4) & 15, n_ar & 15], axis=0)
    dig = jnp.concatenate([dig, jnp.zeros((_DR - 4, N), jnp.int32)], axis=0)
    dig = dig.astype(jnp.float8_e4m3fn)[None]

    nsplit = 4
    hb = B // nsplit
    tbl = tbl.reshape(nsplit, hb * N, cp)
    outs = []
    for h in range(nsplit):
        sl = slice(h * hb, (h + 1) * hb)
        idx = _ball_query_idx(qT[sl], xyz[sl], dig)
        g = _sc_gather(tbl[h], idx.reshape(hb * _NS * M))
        g = g.reshape(hb, _NS, M, cp)
        full = jnp.transpose(g, (0, 3, 2, 1))
        xyzp = full[:, 0:3] - qT[sl, :, :, None]
        outs.append(jnp.concatenate([xyzp, full[:, 3:3 + C]], axis=1))
    return jnp.concatenate(outs, axis=0)
